# Initial kernel scaffold; baseline (speedup 1.0000x reference)
import jax
import jax.numpy as jnp
from jax.experimental import pallas as pl


def kernel(xyz, features, sa_w0, sa_b0, sa_g0, sa_be0, sa_w1, sa_b1, sa_g1, sa_be1, sa_w2, sa_b2, sa_g2, sa_be2, w1, b1, g1, be1, w2, b2, g2, be2, w3, b3, mean_size_arr):
    raise NotImplementedError("write your pallas kernel here")


# speedup vs baseline: 17.5466x; 17.5466x over previous
import functools

import numpy as np
import jax
import jax.numpy as jnp
from jax import lax
from jax.experimental import pallas as pl
from jax.experimental.pallas import tpu as pltpu
from jax.experimental.pallas import tpu_sc as plsc

NUM_CLASS = 18
NUM_HEADING_BIN = 12
NUM_SIZE_CLUSTER = 18
NPOINT = 256
NSAMPLE = 16
RADIUS = 0.3
OUT_CH = 2 + 3 + NUM_HEADING_BIN * 2 + NUM_SIZE_CLUSTER * 4 + NUM_CLASS

_F32 = jnp.float32
_I32 = jnp.int32


def _fps_body(xyzt_ref, out_ref):
    _, b, n = xyzt_ref.shape
    x0 = xyzt_ref[0]
    x1 = xyzt_ref[1]
    x2 = xyzt_ref[2]
    lane = lax.broadcasted_iota(_I32, (b, n), 1)

    def body(i, carry):
        distance, far = carry
        oh = (lane == far).astype(_F32)
        c0 = jnp.sum(x0 * oh, axis=1, keepdims=True)
        c1 = jnp.sum(x1 * oh, axis=1, keepdims=True)
        c2 = jnp.sum(x2 * oh, axis=1, keepdims=True)
        out_ref[pl.ds(i, 1)] = jnp.concatenate([c0, c1, c2], axis=1)[None]
        d0 = x0 - c0
        d1 = x1 - c1
        d2 = x2 - c2
        dist = d0 * d0 + d1 * d1 + d2 * d2
        distance = jnp.minimum(distance, dist)
        m = jnp.max(distance, axis=1, keepdims=True)
        sel = jnp.where(distance == m, lane, n)
        far = jnp.min(sel, axis=1, keepdims=True)
        return distance, far

    lax.fori_loop(
        0, NPOINT, body,
        (jnp.full((b, n), 1e10, _F32), jnp.zeros((b, 1), _I32)))


def _ballq_body(xyzt_ref, new_ref, idx_ref):
    n = xyzt_ref.shape[2]
    bidx = pl.program_id(0)
    nxyz = new_ref[0]
    xk = xyzt_ref[0]
    d2 = jnp.zeros((NPOINT, n), _F32)
    for k in range(3):
        diff = nxyz[:, k:k + 1] - xk[k:k + 1, :]
        d2 = d2 + diff * diff
    mask = d2 < _F32(RADIUS * RADIUS)
    lane = lax.broadcasted_iota(_I32, (NPOINT, n), 1)
    lane_s = lax.broadcasted_iota(_I32, (NPOINT, NSAMPLE), 1)
    idx_acc = jnp.zeros((NPOINT, NSAMPLE), _I32)
    valid = mask
    first0 = None
    for s in range(NSAMPLE):
        cand = jnp.where(valid, lane, n)
        first = jnp.min(cand, axis=1, keepdims=True)
        if s == 0:
            first0 = first
        padded = jnp.where(first == n, first0, first)
        padded = jnp.where(padded == n, 0, padded)
        idx_acc = jnp.where(lane_s == s, padded, idx_acc)
        valid = jnp.logical_and(valid, lane != first)
    idx_ref[0] = idx_acc + bidx * n


def _pre_body(feat_ref, xyz_ref, w0f_ref, w0x_ref, b0_ref, out_ref):
    f = feat_ref[0]
    a = lax.dot_general(f, w0f_ref[...], (((0,), (1,)), ((), ())),
                        preferred_element_type=_F32)
    gx = xyz_ref[0] * _F32(1.0 / RADIUS)
    a = a + lax.dot_general(gx, w0x_ref[...], (((1,), (1,)), ((), ())),
                            preferred_element_type=_F32)
    out_ref[0] = a + b0_ref[...]


def _head_body(grp_ref, new_ref, w0x_ref, g0_ref, be0_ref,
               sw1_ref, sb1_ref, sg1_ref, sbe1_ref,
               sw2_ref, sb2_ref, sg2_ref, sbe2_ref,
               hw1_ref, hb1_ref, hg1_ref, hbe1_ref,
               hw2_ref, hb2_ref, hg2_ref, hbe2_ref,
               hw3_ref, hb3_ref, scale_ref, esel_ref, out_ref):
    def dg(x, w):
        return lax.dot_general(x, w, (((1,), (1,)), ((), ())),
                               preferred_element_type=_F32)

    g = grp_ref[0]
    nx = new_ref[0]
    v = dg(nx * _F32(1.0 / RADIUS), w0x_ref[...])
    vrep = jnp.broadcast_to(
        v[:, None, :], (NPOINT, NSAMPLE, 128)).reshape(NPOINT * NSAMPLE, 128)
    t = g - vrep
    h = jnp.maximum(g0_ref[...] * t + be0_ref[...], 0.0)
    h = jnp.maximum(sg1_ref[...] * (dg(h, sw1_ref[...]) + sb1_ref[...])
                    + sbe1_ref[...], 0.0)
    h = jnp.maximum(sg2_ref[...] * (dg(h, sw2_ref[...]) + sb2_ref[...])
                    + sbe2_ref[...], 0.0)
    pooled = jnp.max(h.reshape(NPOINT, NSAMPLE, 128), axis=1)
    n1 = jnp.maximum(hg1_ref[...] * (dg(pooled, hw1_ref[...]) + hb1_ref[...])
                     + hbe1_ref[...], 0.0)
    n2 = jnp.maximum(hg2_ref[...] * (dg(n1, hw2_ref[...]) + hb2_ref[...])
                     + hbe2_ref[...], 0.0)
    nt = dg(n2, hw3_ref[...]) + hb3_ref[...]
    out_ref[0] = nt * scale_ref[...] + jnp.dot(
        nx, esel_ref[...], preferred_element_type=_F32)


def _sc_gather(table, idx2d):
    rows_tot = idx2d.shape[0]
    n_workers = 32
    per_w = rows_tot // n_workers
    mesh = plsc.VectorSubcoreMesh(core_axis_name="c", subcore_axis_name="s")

    @functools.partial(
        pl.kernel,
        out_type=jax.ShapeDtypeStruct((rows_tot * 128, 128), _F32),
        mesh=mesh,
        scratch_types=[
            pltpu.VMEM((per_w, 128), _I32),
            pltpu.VMEM((128, 128), _F32),
            pltpu.VMEM((128, 128), _F32),
            pltpu.SemaphoreType.DMA,
            pltpu.SemaphoreType.DMA,
        ],
    )
    def gk(table_hbm, idx_hbm, out_hbm, idx_v, buf_a, buf_b, sem_a, sem_b):
        cid = lax.axis_index("c")
        sid = lax.axis_index("s")
        wid = sid * 2 + cid
        base = wid * per_w
        pltpu.sync_copy(idx_hbm.at[pl.ds(base, per_w)], idx_v)
        bufs = (buf_a, buf_b)
        sems = (sem_a, sem_b)
        cps = []
        for j in range(per_w):
            cps.append(pltpu.async_copy(
                table_hbm.at[idx_v.at[j]], bufs[j % 2], sems[j % 2]))
            if j > 0:
                cps[j - 1].wait()
                pltpu.sync_copy(
                    bufs[(j - 1) % 2],
                    out_hbm.at[pl.ds((base + j - 1) * 128, 128)])
        cps[per_w - 1].wait()
        pltpu.sync_copy(
            bufs[(per_w - 1) % 2],
            out_hbm.at[pl.ds((base + per_w - 1) * 128, 128)])

    return gk(table, idx2d)


_ESEL = np.zeros((3, OUT_CH), np.float32)
_ESEL[0, 2] = 1.0
_ESEL[1, 3] = 1.0
_ESEL[2, 4] = 1.0


def kernel(xyz, features, sa_w0, sa_b0, sa_g0, sa_be0, sa_w1, sa_b1, sa_g1,
           sa_be1, sa_w2, sa_b2, sa_g2, sa_be2, w1, b1, g1, be1, w2, b2, g2,
           be2, w3, b3, mean_size_arr):
    b, n, _ = xyz.shape
    c = features.shape[1]

    xyzt = jnp.transpose(xyz, (2, 0, 1))
    new_seq = pl.pallas_call(
        _fps_body,
        out_shape=jax.ShapeDtypeStruct((NPOINT, b, 3), _F32),
    )(xyzt)
    new_xyz = jnp.transpose(new_seq, (1, 0, 2))

    xyzt_b = jnp.transpose(xyz, (0, 2, 1))
    idx = pl.pallas_call(
        _ballq_body,
        grid=(b,),
        in_specs=[
            pl.BlockSpec((1, 3, n), lambda i: (i, 0, 0)),
            pl.BlockSpec((1, NPOINT, 3), lambda i: (i, 0, 0)),
        ],
        out_specs=pl.BlockSpec((1, NPOINT, NSAMPLE), lambda i: (i, 0, 0)),
        out_shape=jax.ShapeDtypeStruct((b, NPOINT, NSAMPLE), _I32),
    )(xyzt_b, new_xyz)

    w0x = sa_w0[:, :3]
    w0f = sa_w0[:, 3:]
    table = pl.pallas_call(
        _pre_body,
        grid=(b,),
        in_specs=[
            pl.BlockSpec((1, c, n), lambda i: (i, 0, 0)),
            pl.BlockSpec((1, n, 3), lambda i: (i, 0, 0)),
            pl.BlockSpec((128, c), lambda i: (0, 0)),
            pl.BlockSpec((128, 3), lambda i: (0, 0)),
            pl.BlockSpec((1, 128), lambda i: (0, 0)),
        ],
        out_specs=pl.BlockSpec((1, n, 128), lambda i: (i, 0, 0)),
        out_shape=jax.ShapeDtypeStruct((b, n, 128), _F32),
    )(features, xyz, w0f, w0x, sa_b0.reshape(1, 128))

    idx2d = idx.reshape(-1, 128)
    grouped = _sc_gather(table.reshape(b * n, 128), idx2d)
    grouped = grouped.reshape(b, NPOINT * NSAMPLE, 128)

    msa_flat = mean_size_arr.reshape(-1)
    scale = jnp.concatenate([
        jnp.ones((5,), _F32),
        jnp.ones((NUM_HEADING_BIN,), _F32),
        jnp.full((NUM_HEADING_BIN,), np.pi / NUM_HEADING_BIN, _F32),
        jnp.ones((NUM_SIZE_CLUSTER,), _F32),
        msa_flat.astype(_F32),
        jnp.ones((NUM_CLASS,), _F32),
    ]).reshape(1, OUT_CH)
    esel = jnp.asarray(_ESEL)

    full = lambda s: pl.BlockSpec(s, lambda i: tuple(0 for _ in s))
    row = lambda: pl.BlockSpec((1, 128), lambda i: (0, 0))
    out = pl.pallas_call(
        _head_body,
        grid=(b,),
        in_specs=[
            pl.BlockSpec((1, NPOINT * NSAMPLE, 128), lambda i: (i, 0, 0)),
            pl.BlockSpec((1, NPOINT, 3), lambda i: (i, 0, 0)),
            full((128, 3)),
            row(), row(),
            full((128, 128)), row(), row(), row(),
            full((128, 128)), row(), row(), row(),
            full((128, 128)), row(), row(), row(),
            full((128, 128)), row(), row(), row(),
            full((OUT_CH, 128)),
            pl.BlockSpec((1, OUT_CH), lambda i: (0, 0)),
            pl.BlockSpec((1, OUT_CH), lambda i: (0, 0)),
            pl.BlockSpec((3, OUT_CH), lambda i: (0, 0)),
        ],
        out_specs=pl.BlockSpec((1, NPOINT, OUT_CH), lambda i: (i, 0, 0)),
        out_shape=jax.ShapeDtypeStruct((b, NPOINT, OUT_CH), _F32),
    )(grouped, new_xyz, w0x,
      sa_g0.reshape(1, 128), sa_be0.reshape(1, 128),
      sa_w1, sa_b1.reshape(1, 128), sa_g1.reshape(1, 128),
      sa_be1.reshape(1, 128),
      sa_w2, sa_b2.reshape(1, 128), sa_g2.reshape(1, 128),
      sa_be2.reshape(1, 128),
      w1, b1.reshape(1, 128), g1.reshape(1, 128), be1.reshape(1, 128),
      w2, b2.reshape(1, 128), g2.reshape(1, 128), be2.reshape(1, 128),
      w3, b3.reshape(1, OUT_CH), scale, esel)
    return out


# speedup vs baseline: 23.2816x; 1.3268x over previous
import functools

import numpy as np
import jax
import jax.numpy as jnp
from jax import lax
from jax.experimental import pallas as pl
from jax.experimental.pallas import tpu as pltpu
from jax.experimental.pallas import tpu_sc as plsc

NUM_CLASS = 18
NUM_HEADING_BIN = 12
NUM_SIZE_CLUSTER = 18
NPOINT = 256
NSAMPLE = 16
RADIUS = 0.3
OUT_CH = 2 + 3 + NUM_HEADING_BIN * 2 + NUM_SIZE_CLUSTER * 4 + NUM_CLASS

_F32 = jnp.float32
_I32 = jnp.int32


def _fps_body(xyzt_ref, out_ref):
    kb, n = xyzt_ref.shape
    b = kb // 3
    x48 = xyzt_ref[...]
    lane48 = lax.broadcasted_iota(_I32, (kb, n), 1)
    lane = lax.broadcasted_iota(_I32, (b, n), 1)

    def body(i, carry):
        distance, far = carry
        far48 = jnp.broadcast_to(far[None], (3, b, 1)).reshape(kb, 1)
        c48 = jnp.sum(jnp.where(lane48 == far48, x48, 0.0), axis=1,
                      keepdims=True)
        out_ref[pl.ds(i, 1)] = c48[None]
        d = x48 - c48
        sq3 = (d * d).reshape(3, b, n)
        dist = sq3[0] + sq3[1] + sq3[2]
        distance = jnp.minimum(distance, dist)
        m = jnp.max(distance, axis=1, keepdims=True)
        sel = jnp.where(distance == m, lane, n)
        far = jnp.min(sel, axis=1, keepdims=True)
        return distance, far

    lax.fori_loop(
        0, NPOINT, body,
        (jnp.full((b, n), 1e10, _F32), jnp.zeros((b, 1), _I32)))


def _bqpre_body(xyzt_ref, new_ref, feat_ref, w0f_ref, w0x_ref, b0_ref,
                idx_ref, tab_ref):
    n = xyzt_ref.shape[2]
    bidx = pl.program_id(0)
    x3n = xyzt_ref[0]
    nb = new_ref[0]

    a = lax.dot_general(feat_ref[0], w0f_ref[...], (((0,), (1,)), ((), ())),
                        preferred_element_type=_F32)
    a = a + lax.dot_general(x3n * _F32(1.0 / RADIUS), w0x_ref[...],
                            (((0,), (1,)), ((), ())),
                            preferred_element_type=_F32)
    tab_ref[0] = a + b0_ref[...]

    d2 = jnp.zeros((NPOINT, n), _F32)
    for k in range(3):
        diff = nb[:, k:k + 1] - x3n[k:k + 1, :]
        d2 = d2 + diff * diff
    lane = lax.broadcasted_iota(_I32, (NPOINT, n), 1)
    lane_s = lax.broadcasted_iota(_I32, (NPOINT, NSAMPLE), 1)
    work = jnp.where(d2 < _F32(RADIUS * RADIUS), lane, n)
    first0 = jnp.min(work, axis=1, keepdims=True)
    pad0 = jnp.where(first0 == n, 0, first0)
    acc = jnp.broadcast_to(pad0, (NPOINT, NSAMPLE))
    work = jnp.where(work == first0, n, work)

    def cond(st):
        return st[3]

    def body(st):
        s, work, acc, _ = st
        first = jnp.min(work, axis=1, keepdims=True)
        acc = jnp.where(jnp.logical_and(lane_s == s, first < n), first, acc)
        work = jnp.where(work == first, n, work)
        alive = jnp.logical_and(s + 1 < NSAMPLE, jnp.min(first) < n)
        return (s + 1, work, acc, alive)

    alive0 = jnp.logical_and(1 < NSAMPLE, jnp.min(work) < n)
    _, _, acc, _ = lax.while_loop(
        cond, body, (jnp.int32(1), work, acc, alive0))
    idx_ref[0] = acc + bidx * n


def _head_body(grp_ref, new_ref, w0x_ref,
               sw1_ref, sb1_ref, sw2_ref, sb2_ref,
               hw1_ref, hb1_ref, hw2_ref, hb2_ref,
               hw3_ref, hb3_ref, scale_ref, esel_ref, out_ref):
    def dg(x, w):
        return lax.dot_general(x, w, (((1,), (1,)), ((), ())),
                               preferred_element_type=_F32)

    g = grp_ref[0]
    nx = new_ref[0]
    v = dg(nx * _F32(1.0 / RADIUS), w0x_ref[...])
    vrep = jnp.broadcast_to(
        v[:, None, :], (NPOINT, NSAMPLE, 128)).reshape(NPOINT * NSAMPLE, 128)
    h = jnp.maximum(g - vrep, 0.0)
    h = jnp.maximum(dg(h, sw1_ref[...]) + sb1_ref[...], 0.0)
    h = jnp.maximum(dg(h, sw2_ref[...]) + sb2_ref[...], 0.0)
    pooled = jnp.max(h.reshape(NPOINT, NSAMPLE, 128), axis=1)
    n1 = jnp.maximum(dg(pooled, hw1_ref[...]) + hb1_ref[...], 0.0)
    n2 = jnp.maximum(dg(n1, hw2_ref[...]) + hb2_ref[...], 0.0)
    nt = dg(n2, hw3_ref[...]) + hb3_ref[...]
    out_ref[0] = nt * scale_ref[...] + jnp.dot(
        nx, esel_ref[...], preferred_element_type=_F32)


def _sc_gather(table, idx2d):
    rows_tot = idx2d.shape[0]
    n_workers = 32
    per_w = rows_tot // n_workers
    mesh = plsc.VectorSubcoreMesh(core_axis_name="c", subcore_axis_name="s")

    @functools.partial(
        pl.kernel,
        out_type=jax.ShapeDtypeStruct((rows_tot * 128, 128), _F32),
        mesh=mesh,
        scratch_types=[
            pltpu.VMEM((per_w, 128), _I32),
            pltpu.VMEM((128, 128), _F32),
            pltpu.VMEM((128, 128), _F32),
            pltpu.SemaphoreType.DMA,
            pltpu.SemaphoreType.DMA,
        ],
    )
    def gk(table_hbm, idx_hbm, out_hbm, idx_v, buf_a, buf_b, sem_a, sem_b):
        cid = lax.axis_index("c")
        sid = lax.axis_index("s")
        wid = sid * 2 + cid
        base = wid * per_w
        pltpu.sync_copy(idx_hbm.at[pl.ds(base, per_w)], idx_v)
        bufs = (buf_a, buf_b)
        sems = (sem_a, sem_b)
        cps = []
        for j in range(per_w):
            cps.append(pltpu.async_copy(
                table_hbm.at[idx_v.at[j]], bufs[j % 2], sems[j % 2]))
            if j > 0:
                cps[j - 1].wait()
                pltpu.sync_copy(
                    bufs[(j - 1) % 2],
                    out_hbm.at[pl.ds((base + j - 1) * 128, 128)])
        cps[per_w - 1].wait()
        pltpu.sync_copy(
            bufs[(per_w - 1) % 2],
            out_hbm.at[pl.ds((base + per_w - 1) * 128, 128)])

    return gk(table, idx2d)


_ESEL = np.zeros((3, OUT_CH), np.float32)
_ESEL[0, 2] = 1.0
_ESEL[1, 3] = 1.0
_ESEL[2, 4] = 1.0


def kernel(xyz, features, sa_w0, sa_b0, sa_g0, sa_be0, sa_w1, sa_b1, sa_g1,
           sa_be1, sa_w2, sa_b2, sa_g2, sa_be2, w1, b1, g1, be1, w2, b2, g2,
           be2, w3, b3, mean_size_arr):
    b, n, _ = xyz.shape
    c = features.shape[1]

    w0e = sa_w0 * sa_g0[:, None]
    b0e = sa_b0 * sa_g0 + sa_be0
    w0x = w0e[:, :3]
    w0f = w0e[:, 3:]
    sw1 = sa_w1 * sa_g1[:, None]
    sb1 = sa_b1 * sa_g1 + sa_be1
    sw2 = sa_w2 * sa_g2[:, None]
    sb2 = sa_b2 * sa_g2 + sa_be2
    hw1 = w1 * g1[:, None]
    hb1 = b1 * g1 + be1
    hw2 = w2 * g2[:, None]
    hb2 = b2 * g2 + be2

    xyzt = jnp.transpose(xyz, (2, 0, 1)).reshape(3 * b, n)
    new_seq = pl.pallas_call(
        _fps_body,
        out_shape=jax.ShapeDtypeStruct((NPOINT, 3 * b, 1), _F32),
    )(xyzt)
    new_xyz = jnp.transpose(new_seq.reshape(NPOINT, 3, b), (2, 0, 1))

    xyzt_b = jnp.transpose(xyz, (0, 2, 1))
    idx, table = pl.pallas_call(
        _bqpre_body,
        grid=(b,),
        in_specs=[
            pl.BlockSpec((1, 3, n), lambda i: (i, 0, 0)),
            pl.BlockSpec((1, NPOINT, 3), lambda i: (i, 0, 0)),
            pl.BlockSpec((1, c, n), lambda i: (i, 0, 0)),
            pl.BlockSpec((128, c), lambda i: (0, 0)),
            pl.BlockSpec((128, 3), lambda i: (0, 0)),
            pl.BlockSpec((1, 128), lambda i: (0, 0)),
        ],
        out_specs=[
            pl.BlockSpec((1, NPOINT, NSAMPLE), lambda i: (i, 0, 0)),
            pl.BlockSpec((1, n, 128), lambda i: (i, 0, 0)),
        ],
        out_shape=[
            jax.ShapeDtypeStruct((b, NPOINT, NSAMPLE), _I32),
            jax.ShapeDtypeStruct((b, n, 128), _F32),
        ],
    )(xyzt_b, new_xyz, features, w0f, w0x, b0e.reshape(1, 128))

    idx2d = idx.reshape(-1, 128)
    grouped = _sc_gather(table.reshape(b * n, 128), idx2d)
    grouped = grouped.reshape(b, NPOINT * NSAMPLE, 128)

    msa_flat = mean_size_arr.reshape(-1)
    scale = jnp.concatenate([
        jnp.ones((5,), _F32),
        jnp.ones((NUM_HEADING_BIN,), _F32),
        jnp.full((NUM_HEADING_BIN,), np.pi / NUM_HEADING_BIN, _F32),
        jnp.ones((NUM_SIZE_CLUSTER,), _F32),
        msa_flat.astype(_F32),
        jnp.ones((NUM_CLASS,), _F32),
    ]).reshape(1, OUT_CH)
    esel = jnp.asarray(_ESEL)

    full = lambda s: pl.BlockSpec(s, lambda i: tuple(0 for _ in s))
    row = lambda: pl.BlockSpec((1, 128), lambda i: (0, 0))
    out = pl.pallas_call(
        _head_body,
        grid=(b,),
        in_specs=[
            pl.BlockSpec((1, NPOINT * NSAMPLE, 128), lambda i: (i, 0, 0)),
            pl.BlockSpec((1, NPOINT, 3), lambda i: (i, 0, 0)),
            full((128, 3)),
            full((128, 128)), row(),
            full((128, 128)), row(),
            full((128, 128)), row(),
            full((128, 128)), row(),
            full((OUT_CH, 128)),
            pl.BlockSpec((1, OUT_CH), lambda i: (0, 0)),
            pl.BlockSpec((1, OUT_CH), lambda i: (0, 0)),
            pl.BlockSpec((3, OUT_CH), lambda i: (0, 0)),
        ],
        out_specs=pl.BlockSpec((1, NPOINT, OUT_CH), lambda i: (i, 0, 0)),
        out_shape=jax.ShapeDtypeStruct((b, NPOINT, OUT_CH), _F32),
    )(grouped, new_xyz, w0x,
      sw1, sb1.reshape(1, 128),
      sw2, sb2.reshape(1, 128),
      hw1, hb1.reshape(1, 128),
      hw2, hb2.reshape(1, 128),
      w3, b3.reshape(1, OUT_CH), scale, esel)
    return out


# speedup vs baseline: 23.5916x; 1.0133x over previous
import functools

import numpy as np
import jax
import jax.numpy as jnp
from jax import lax
from jax.experimental import pallas as pl
from jax.experimental.pallas import tpu as pltpu
from jax.experimental.pallas import tpu_sc as plsc

NUM_CLASS = 18
NUM_HEADING_BIN = 12
NUM_SIZE_CLUSTER = 18
NPOINT = 256
NSAMPLE = 16
RADIUS = 0.3
OUT_CH = 2 + 3 + NUM_HEADING_BIN * 2 + NUM_SIZE_CLUSTER * 4 + NUM_CLASS

_F32 = jnp.float32
_I32 = jnp.int32


def _fps_body(xyzt_ref, out_ref):
    kb, n = xyzt_ref.shape
    b = kb // 3
    x48 = xyzt_ref[...]
    lane48 = lax.broadcasted_iota(_I32, (kb, n), 1)
    lane = lax.broadcasted_iota(_I32, (b, n), 1)
    lane_p = lax.broadcasted_iota(_I32, (kb, NPOINT), 1)

    def body(i, carry):
        distance, far, acc = carry
        far48 = jnp.broadcast_to(far[None], (3, b, 1)).reshape(kb, 1)
        c48 = jnp.sum(jnp.where(lane48 == far48, x48, 0.0), axis=1,
                      keepdims=True)
        acc = jnp.where(lane_p == i, c48, acc)
        d = x48 - c48
        sq3 = (d * d).reshape(3, b, n)
        dist = sq3[0] + sq3[1] + sq3[2]
        distance = jnp.minimum(distance, dist)
        m = jnp.max(distance, axis=1, keepdims=True)
        sel = jnp.where(distance == m, lane, n)
        far = jnp.min(sel, axis=1, keepdims=True)
        return distance, far, acc

    _, _, acc = lax.fori_loop(
        0, NPOINT, body,
        (jnp.full((b, n), 1e10, _F32), jnp.zeros((b, 1), _I32),
         jnp.zeros((kb, NPOINT), _F32)))
    out_ref[...] = acc


def _bqpre_body(xyzt_ref, new_ref, feat_ref, w0f_ref, w0x_ref, b0_ref,
                idx_ref, tab_ref):
    n = xyzt_ref.shape[2]
    bidx = pl.program_id(0)
    x3n = xyzt_ref[0]
    nb = new_ref[0]

    a = lax.dot_general(feat_ref[0], w0f_ref[...], (((0,), (1,)), ((), ())),
                        preferred_element_type=_F32)
    a = a + lax.dot_general(x3n * _F32(1.0 / RADIUS), w0x_ref[...],
                            (((0,), (1,)), ((), ())),
                            preferred_element_type=_F32)
    tab_ref[0] = a + b0_ref[...]

    d2 = jnp.zeros((NPOINT, n), _F32)
    for k in range(3):
        diff = nb[:, k:k + 1] - x3n[k:k + 1, :]
        d2 = d2 + diff * diff
    lane = lax.broadcasted_iota(_I32, (NPOINT, n), 1)
    lane_s = lax.broadcasted_iota(_I32, (NPOINT, NSAMPLE), 1)
    work = jnp.where(d2 < _F32(RADIUS * RADIUS), lane, n)
    first0 = jnp.min(work, axis=1, keepdims=True)
    pad0 = jnp.where(first0 == n, 0, first0)
    acc = jnp.broadcast_to(pad0, (NPOINT, NSAMPLE))
    work = jnp.where(work == first0, n, work)

    def cond(st):
        return st[3]

    def body(st):
        s, work, acc, _ = st
        first = jnp.min(work, axis=1, keepdims=True)
        acc = jnp.where(jnp.logical_and(lane_s == s, first < n), first, acc)
        work = jnp.where(work == first, n, work)
        alive = jnp.logical_and(s + 1 < NSAMPLE, jnp.min(first) < n)
        return (s + 1, work, acc, alive)

    alive0 = jnp.logical_and(1 < NSAMPLE, jnp.min(work) < n)
    _, _, acc, _ = lax.while_loop(
        cond, body, (jnp.int32(1), work, acc, alive0))
    idx_ref[0] = acc + bidx * n


def _head_body(grp_ref, new_ref, w0x_ref,
               sw1_ref, sb1_ref, sw2_ref, sb2_ref,
               hw1_ref, hb1_ref, hw2_ref, hb2_ref,
               hw3_ref, hb3_ref, scale_ref, esel_ref, out_ref):
    def dg(x, w):
        return lax.dot_general(x, w, (((1,), (1,)), ((), ())),
                               preferred_element_type=_F32)

    g = grp_ref[0]
    nx = new_ref[0]
    v = dg(nx * _F32(1.0 / RADIUS), w0x_ref[...])
    vrep = jnp.broadcast_to(
        v[:, None, :], (NPOINT, NSAMPLE, 128)).reshape(NPOINT * NSAMPLE, 128)
    h = jnp.maximum(g - vrep, 0.0)
    h = jnp.maximum(dg(h, sw1_ref[...]) + sb1_ref[...], 0.0)
    h = jnp.maximum(dg(h, sw2_ref[...]) + sb2_ref[...], 0.0)
    pooled = jnp.max(h.reshape(NPOINT, NSAMPLE, 128), axis=1)
    n1 = jnp.maximum(dg(pooled, hw1_ref[...]) + hb1_ref[...], 0.0)
    n2 = jnp.maximum(dg(n1, hw2_ref[...]) + hb2_ref[...], 0.0)
    nt = dg(n2, hw3_ref[...]) + hb3_ref[...]
    out_ref[0] = nt * scale_ref[...] + jnp.dot(
        nx, esel_ref[...], preferred_element_type=_F32)


def _sc_gather(table, idx2d):
    rows_tot = idx2d.shape[0]
    n_workers = 32
    per_w = rows_tot // n_workers
    mesh = plsc.VectorSubcoreMesh(core_axis_name="c", subcore_axis_name="s")

    @functools.partial(
        pl.kernel,
        out_type=jax.ShapeDtypeStruct((rows_tot * 128, 128), _F32),
        mesh=mesh,
        scratch_types=[
            pltpu.VMEM((per_w, 128), _I32),
            pltpu.VMEM((128, 128), _F32),
            pltpu.VMEM((128, 128), _F32),
            pltpu.SemaphoreType.DMA,
            pltpu.SemaphoreType.DMA,
        ],
    )
    def gk(table_hbm, idx_hbm, out_hbm, idx_v, buf_a, buf_b, sem_a, sem_b):
        cid = lax.axis_index("c")
        sid = lax.axis_index("s")
        wid = sid * 2 + cid
        base = wid * per_w
        pltpu.sync_copy(idx_hbm.at[pl.ds(base, per_w)], idx_v)
        bufs = (buf_a, buf_b)
        sems = (sem_a, sem_b)
        cps = []
        for j in range(per_w):
            cps.append(pltpu.async_copy(
                table_hbm.at[idx_v.at[j]], bufs[j % 2], sems[j % 2]))
            if j > 0:
                cps[j - 1].wait()
                pltpu.sync_copy(
                    bufs[(j - 1) % 2],
                    out_hbm.at[pl.ds((base + j - 1) * 128, 128)])
        cps[per_w - 1].wait()
        pltpu.sync_copy(
            bufs[(per_w - 1) % 2],
            out_hbm.at[pl.ds((base + per_w - 1) * 128, 128)])

    return gk(table, idx2d)


_ESEL = np.zeros((3, OUT_CH), np.float32)
_ESEL[0, 2] = 1.0
_ESEL[1, 3] = 1.0
_ESEL[2, 4] = 1.0


def kernel(xyz, features, sa_w0, sa_b0, sa_g0, sa_be0, sa_w1, sa_b1, sa_g1,
           sa_be1, sa_w2, sa_b2, sa_g2, sa_be2, w1, b1, g1, be1, w2, b2, g2,
           be2, w3, b3, mean_size_arr):
    b, n, _ = xyz.shape
    c = features.shape[1]

    w0e = sa_w0 * sa_g0[:, None]
    b0e = sa_b0 * sa_g0 + sa_be0
    w0x = w0e[:, :3]
    w0f = w0e[:, 3:]
    sw1 = sa_w1 * sa_g1[:, None]
    sb1 = sa_b1 * sa_g1 + sa_be1
    sw2 = sa_w2 * sa_g2[:, None]
    sb2 = sa_b2 * sa_g2 + sa_be2
    hw1 = w1 * g1[:, None]
    hb1 = b1 * g1 + be1
    hw2 = w2 * g2[:, None]
    hb2 = b2 * g2 + be2

    xyzt = jnp.transpose(xyz, (2, 0, 1)).reshape(3 * b, n)
    new_seq = pl.pallas_call(
        _fps_body,
        out_shape=jax.ShapeDtypeStruct((3 * b, NPOINT), _F32),
    )(xyzt)
    new_xyz = jnp.transpose(new_seq.reshape(3, b, NPOINT), (1, 2, 0))

    xyzt_b = jnp.transpose(xyz, (0, 2, 1))
    idx, table = pl.pallas_call(
        _bqpre_body,
        grid=(b,),
        in_specs=[
            pl.BlockSpec((1, 3, n), lambda i: (i, 0, 0)),
            pl.BlockSpec((1, NPOINT, 3), lambda i: (i, 0, 0)),
            pl.BlockSpec((1, c, n), lambda i: (i, 0, 0)),
            pl.BlockSpec((128, c), lambda i: (0, 0)),
            pl.BlockSpec((128, 3), lambda i: (0, 0)),
            pl.BlockSpec((1, 128), lambda i: (0, 0)),
        ],
        out_specs=[
            pl.BlockSpec((1, NPOINT, NSAMPLE), lambda i: (i, 0, 0)),
            pl.BlockSpec((1, n, 128), lambda i: (i, 0, 0)),
        ],
        out_shape=[
            jax.ShapeDtypeStruct((b, NPOINT, NSAMPLE), _I32),
            jax.ShapeDtypeStruct((b, n, 128), _F32),
        ],
    )(xyzt_b, new_xyz, features, w0f, w0x, b0e.reshape(1, 128))

    idx2d = idx.reshape(-1, 128)
    table_flat = table.reshape(b * n, 128)
    half_rows = idx2d.shape[0] // 2
    hb = b // 2
    grouped_halves = [
        _sc_gather(table_flat, idx2d[h * half_rows:(h + 1) * half_rows])
        .reshape(hb, NPOINT * NSAMPLE, 128)
        for h in range(2)
    ]

    msa_flat = mean_size_arr.reshape(-1)
    scale = jnp.concatenate([
        jnp.ones((5,), _F32),
        jnp.ones((NUM_HEADING_BIN,), _F32),
        jnp.full((NUM_HEADING_BIN,), np.pi / NUM_HEADING_BIN, _F32),
        jnp.ones((NUM_SIZE_CLUSTER,), _F32),
        msa_flat.astype(_F32),
        jnp.ones((NUM_CLASS,), _F32),
    ]).reshape(1, OUT_CH)
    esel = jnp.asarray(_ESEL)

    full = lambda s: pl.BlockSpec(s, lambda i: tuple(0 for _ in s))
    row = lambda: pl.BlockSpec((1, 128), lambda i: (0, 0))

    def run_head(grouped_h, new_h):
        bh = grouped_h.shape[0]
        return pl.pallas_call(
            _head_body,
            grid=(bh,),
            in_specs=[
                pl.BlockSpec((1, NPOINT * NSAMPLE, 128), lambda i: (i, 0, 0)),
                pl.BlockSpec((1, NPOINT, 3), lambda i: (i, 0, 0)),
                full((128, 3)),
                full((128, 128)), row(),
                full((128, 128)), row(),
                full((128, 128)), row(),
                full((128, 128)), row(),
                full((OUT_CH, 128)),
                pl.BlockSpec((1, OUT_CH), lambda i: (0, 0)),
                pl.BlockSpec((1, OUT_CH), lambda i: (0, 0)),
                pl.BlockSpec((3, OUT_CH), lambda i: (0, 0)),
            ],
            out_specs=pl.BlockSpec((1, NPOINT, OUT_CH), lambda i: (i, 0, 0)),
            out_shape=jax.ShapeDtypeStruct((bh, NPOINT, OUT_CH), _F32),
        )(grouped_h, new_h, w0x,
          sw1, sb1.reshape(1, 128),
          sw2, sb2.reshape(1, 128),
          hw1, hb1.reshape(1, 128),
          hw2, hb2.reshape(1, 128),
          w3, b3.reshape(1, OUT_CH), scale, esel)

    out_halves = [
        run_head(grouped_halves[h], new_xyz[h * hb:(h + 1) * hb])
        for h in range(2)
    ]
    return jnp.concatenate(out_halves, axis=0)


# speedup vs baseline: 23.7591x; 1.0071x over previous
import functools

import numpy as np
import jax
import jax.numpy as jnp
from jax import lax
from jax.experimental import pallas as pl
from jax.experimental.pallas import tpu as pltpu
from jax.experimental.pallas import tpu_sc as plsc

NUM_CLASS = 18
NUM_HEADING_BIN = 12
NUM_SIZE_CLUSTER = 18
NPOINT = 256
NSAMPLE = 16
RADIUS = 0.3
OUT_CH = 2 + 3 + NUM_HEADING_BIN * 2 + NUM_SIZE_CLUSTER * 4 + NUM_CLASS

_F32 = jnp.float32
_I32 = jnp.int32


def _fps_body(xyzt_ref, out_ref):
    kb, n = xyzt_ref.shape
    b = kb // 3
    x48 = xyzt_ref[...]
    lane48 = lax.broadcasted_iota(_I32, (kb, n), 1)
    lane = lax.broadcasted_iota(_I32, (b, n), 1)
    lane_p = lax.broadcasted_iota(_I32, (kb, NPOINT), 1)

    def body(i, carry):
        distance, far, acc = carry
        far48 = jnp.broadcast_to(far[None], (3, b, 1)).reshape(kb, 1)
        c48 = jnp.sum(jnp.where(lane48 == far48, x48, 0.0), axis=1,
                      keepdims=True)
        acc = jnp.where(lane_p == i, c48, acc)
        d = x48 - c48
        sq3 = (d * d).reshape(3, b, n)
        dist = sq3[0] + sq3[1] + sq3[2]
        distance = jnp.minimum(distance, dist)
        m = jnp.max(distance, axis=1, keepdims=True)
        sel = jnp.where(distance == m, lane, n)
        far = jnp.min(sel, axis=1, keepdims=True)
        return distance, far, acc

    _, _, acc = lax.fori_loop(
        0, NPOINT, body,
        (jnp.full((b, n), 1e10, _F32), jnp.zeros((b, 1), _I32),
         jnp.zeros((kb, NPOINT), _F32)))
    out_ref[...] = acc


def _bqpre_body(xyzt_ref, new_ref, feat_ref, w0f_ref, w0x_ref, b0_ref,
                idx_ref, tab_ref):
    n = xyzt_ref.shape[2]
    bidx = pl.program_id(0)
    x3n = xyzt_ref[0]
    nb = new_ref[0]

    a = lax.dot_general(feat_ref[0], w0f_ref[...], (((0,), (1,)), ((), ())),
                        preferred_element_type=_F32)
    a = a + lax.dot_general(x3n * _F32(1.0 / RADIUS), w0x_ref[...],
                            (((0,), (1,)), ((), ())),
                            preferred_element_type=_F32)
    tab_ref[0] = a + b0_ref[...]

    d2 = jnp.zeros((NPOINT, n), _F32)
    for k in range(3):
        diff = nb[:, k:k + 1] - x3n[k:k + 1, :]
        d2 = d2 + diff * diff
    lane = lax.broadcasted_iota(_I32, (NPOINT, n), 1)
    lane_s = lax.broadcasted_iota(_I32, (NPOINT, NSAMPLE), 1)
    work = jnp.where(d2 < _F32(RADIUS * RADIUS), lane, n)
    first0 = jnp.min(work, axis=1, keepdims=True)
    pad0 = jnp.where(first0 == n, 0, first0)
    acc = jnp.broadcast_to(pad0, (NPOINT, NSAMPLE))
    work = jnp.where(work == first0, n, work)

    def cond(st):
        return st[3]

    def body(st):
        s, work, acc, _ = st
        first = jnp.min(work, axis=1, keepdims=True)
        acc = jnp.where(jnp.logical_and(lane_s == s, first < n), first, acc)
        work = jnp.where(work == first, n, work)
        alive = jnp.logical_and(s + 1 < NSAMPLE, jnp.min(first) < n)
        return (s + 1, work, acc, alive)

    alive0 = jnp.logical_and(1 < NSAMPLE, jnp.min(work) < n)
    _, _, acc, _ = lax.while_loop(
        cond, body, (jnp.int32(1), work, acc, alive0))
    idx_ref[0] = acc + bidx * n


def _head_body(grp_ref, new_ref, w0x_ref,
               sw1_ref, sb1_ref, sw2_ref, sb2_ref,
               hw1_ref, hb1_ref, hw2_ref, hb2_ref,
               hw3_ref, hb3_ref, scale_ref, esel_ref, out_ref):
    def dg(x, w):
        return lax.dot_general(x, w, (((1,), (1,)), ((), ())),
                               preferred_element_type=_F32)

    g = grp_ref[0]
    nx = new_ref[0]
    v = dg(nx * _F32(1.0 / RADIUS), w0x_ref[...])
    vrep = jnp.broadcast_to(
        v[:, None, :], (NPOINT, NSAMPLE, 128)).reshape(NPOINT * NSAMPLE, 128)
    h = jnp.maximum(g - vrep, 0.0)
    h = jnp.maximum(dg(h, sw1_ref[...]) + sb1_ref[...], 0.0)
    h = jnp.maximum(dg(h, sw2_ref[...]) + sb2_ref[...], 0.0)
    pooled = jnp.max(h.reshape(NPOINT, NSAMPLE, 128), axis=1)
    n1 = jnp.maximum(dg(pooled, hw1_ref[...]) + hb1_ref[...], 0.0)
    n2 = jnp.maximum(dg(n1, hw2_ref[...]) + hb2_ref[...], 0.0)
    nt = dg(n2, hw3_ref[...]) + hb3_ref[...]
    out_ref[0] = nt * scale_ref[...] + jnp.dot(
        nx, esel_ref[...], preferred_element_type=_F32)


def _sc_gather(table, idx2d):
    rows_tot = idx2d.shape[0]
    n_workers = 32
    per_w = rows_tot // n_workers
    nb = 3
    dt = table.dtype
    mesh = plsc.VectorSubcoreMesh(core_axis_name="c", subcore_axis_name="s")

    @functools.partial(
        pl.kernel,
        out_type=jax.ShapeDtypeStruct((rows_tot * 128, 128), dt),
        mesh=mesh,
        scratch_types=(
            [pltpu.VMEM((per_w, 128), _I32)]
            + [pltpu.VMEM((128, 128), dt) for _ in range(nb)]
            + [pltpu.SemaphoreType.DMA for _ in range(2 * nb)]
        ),
    )
    def gk(table_hbm, idx_hbm, out_hbm, idx_v, *bs):
        bufs = bs[:nb]
        gsem = bs[nb:2 * nb]
        ssem = bs[2 * nb:]
        cid = lax.axis_index("c")
        sid = lax.axis_index("s")
        wid = sid * 2 + cid
        base = wid * per_w
        pltpu.sync_copy(idx_hbm.at[pl.ds(base, per_w)], idx_v)
        gcp = [None] * per_w
        scp = [None] * per_w

        def fire_store(j):
            scp[j] = pltpu.async_copy(
                bufs[j % nb],
                out_hbm.at[pl.ds((base + j) * 128, 128)], ssem[j % nb])

        for j in range(per_w):
            if j >= nb:
                scp[j - nb].wait()
            gcp[j] = pltpu.async_copy(
                table_hbm.at[idx_v.at[j]], bufs[j % nb], gsem[j % nb])
            if j > 0:
                gcp[j - 1].wait()
                fire_store(j - 1)
        gcp[per_w - 1].wait()
        fire_store(per_w - 1)
        for j in range(max(per_w - nb, 0), per_w):
            scp[j].wait()

    return gk(table, idx2d)


_ESEL = np.zeros((3, OUT_CH), np.float32)
_ESEL[0, 2] = 1.0
_ESEL[1, 3] = 1.0
_ESEL[2, 4] = 1.0


def kernel(xyz, features, sa_w0, sa_b0, sa_g0, sa_be0, sa_w1, sa_b1, sa_g1,
           sa_be1, sa_w2, sa_b2, sa_g2, sa_be2, w1, b1, g1, be1, w2, b2, g2,
           be2, w3, b3, mean_size_arr):
    b, n, _ = xyz.shape
    c = features.shape[1]

    w0e = sa_w0 * sa_g0[:, None]
    b0e = sa_b0 * sa_g0 + sa_be0
    w0x = w0e[:, :3]
    w0f = w0e[:, 3:]
    sw1 = sa_w1 * sa_g1[:, None]
    sb1 = sa_b1 * sa_g1 + sa_be1
    sw2 = sa_w2 * sa_g2[:, None]
    sb2 = sa_b2 * sa_g2 + sa_be2
    hw1 = w1 * g1[:, None]
    hb1 = b1 * g1 + be1
    hw2 = w2 * g2[:, None]
    hb2 = b2 * g2 + be2

    xyzt = jnp.transpose(xyz, (2, 0, 1)).reshape(3 * b, n)
    new_seq = pl.pallas_call(
        _fps_body,
        out_shape=jax.ShapeDtypeStruct((3 * b, NPOINT), _F32),
    )(xyzt)
    new_xyz = jnp.transpose(new_seq.reshape(3, b, NPOINT), (1, 2, 0))

    xyzt_b = jnp.transpose(xyz, (0, 2, 1))
    idx, table = pl.pallas_call(
        _bqpre_body,
        grid=(b,),
        in_specs=[
            pl.BlockSpec((1, 3, n), lambda i: (i, 0, 0)),
            pl.BlockSpec((1, NPOINT, 3), lambda i: (i, 0, 0)),
            pl.BlockSpec((1, c, n), lambda i: (i, 0, 0)),
            pl.BlockSpec((128, c), lambda i: (0, 0)),
            pl.BlockSpec((128, 3), lambda i: (0, 0)),
            pl.BlockSpec((1, 128), lambda i: (0, 0)),
        ],
        out_specs=[
            pl.BlockSpec((1, NPOINT, NSAMPLE), lambda i: (i, 0, 0)),
            pl.BlockSpec((1, n, 128), lambda i: (i, 0, 0)),
        ],
        out_shape=[
            jax.ShapeDtypeStruct((b, NPOINT, NSAMPLE), _I32),
            jax.ShapeDtypeStruct((b, n, 128), _F32),
        ],
    )(xyzt_b, new_xyz, features, w0f, w0x, b0e.reshape(1, 128))

    idx2d = idx.reshape(-1, 128)
    table_flat = table.reshape(b * n, 128)
    half_rows = idx2d.shape[0] // 2
    hb = b // 2
    grouped_halves = [
        _sc_gather(table_flat, idx2d[h * half_rows:(h + 1) * half_rows])
        .reshape(hb, NPOINT * NSAMPLE, 128)
        for h in range(2)
    ]

    msa_flat = mean_size_arr.reshape(-1)
    scale = jnp.concatenate([
        jnp.ones((5,), _F32),
        jnp.ones((NUM_HEADING_BIN,), _F32),
        jnp.full((NUM_HEADING_BIN,), np.pi / NUM_HEADING_BIN, _F32),
        jnp.ones((NUM_SIZE_CLUSTER,), _F32),
        msa_flat.astype(_F32),
        jnp.ones((NUM_CLASS,), _F32),
    ]).reshape(1, OUT_CH)
    esel = jnp.asarray(_ESEL)

    full = lambda s: pl.BlockSpec(s, lambda i: tuple(0 for _ in s))
    row = lambda: pl.BlockSpec((1, 128), lambda i: (0, 0))

    def run_head(grouped_h, new_h):
        bh = grouped_h.shape[0]
        return pl.pallas_call(
            _head_body,
            grid=(bh,),
            in_specs=[
                pl.BlockSpec((1, NPOINT * NSAMPLE, 128), lambda i: (i, 0, 0)),
                pl.BlockSpec((1, NPOINT, 3), lambda i: (i, 0, 0)),
                full((128, 3)),
                full((128, 128)), row(),
                full((128, 128)), row(),
                full((128, 128)), row(),
                full((128, 128)), row(),
                full((OUT_CH, 128)),
                pl.BlockSpec((1, OUT_CH), lambda i: (0, 0)),
                pl.BlockSpec((1, OUT_CH), lambda i: (0, 0)),
                pl.BlockSpec((3, OUT_CH), lambda i: (0, 0)),
            ],
            out_specs=pl.BlockSpec((1, NPOINT, OUT_CH), lambda i: (i, 0, 0)),
            out_shape=jax.ShapeDtypeStruct((bh, NPOINT, OUT_CH), _F32),
        )(grouped_h, new_h, w0x,
          sw1, sb1.reshape(1, 128),
          sw2, sb2.reshape(1, 128),
          hw1, hb1.reshape(1, 128),
          hw2, hb2.reshape(1, 128),
          w3, b3.reshape(1, OUT_CH), scale, esel)

    out_halves = [
        run_head(grouped_halves[h], new_xyz[h * hb:(h + 1) * hb])
        for h in range(2)
    ]
    return jnp.concatenate(out_halves, axis=0)


# speedup vs baseline: 25.0088x; 1.0526x over previous
import functools

import numpy as np
import jax
import jax.numpy as jnp
from jax import lax
from jax.experimental import pallas as pl
from jax.experimental.pallas import tpu as pltpu
from jax.experimental.pallas import tpu_sc as plsc

NUM_CLASS = 18
NUM_HEADING_BIN = 12
NUM_SIZE_CLUSTER = 18
NPOINT = 256
NSAMPLE = 16
RADIUS = 0.3
OUT_CH = 2 + 3 + NUM_HEADING_BIN * 2 + NUM_SIZE_CLUSTER * 4 + NUM_CLASS

_F32 = jnp.float32
_I32 = jnp.int32


def _fps_body(xyzt_ref, out_ref):
    kb, n = xyzt_ref.shape
    b = kb // 3
    x48 = xyzt_ref[...]
    lane48 = lax.broadcasted_iota(_I32, (kb, n), 1)
    lane = lax.broadcasted_iota(_I32, (b, n), 1)
    lane_p = lax.broadcasted_iota(_I32, (kb, NPOINT), 1)

    def body(i, carry):
        distance, far, acc = carry
        far48 = jnp.broadcast_to(far[None], (3, b, 1)).reshape(kb, 1)
        c48 = jnp.sum(jnp.where(lane48 == far48, x48, 0.0), axis=1,
                      keepdims=True)
        acc = jnp.where(lane_p == i, c48, acc)
        d = x48 - c48
        sq3 = (d * d).reshape(3, b, n)
        dist = sq3[0] + sq3[1] + sq3[2]
        distance = jnp.minimum(distance, dist)
        m = jnp.max(distance, axis=1, keepdims=True)
        sel = jnp.where(distance == m, lane, n)
        far = jnp.min(sel, axis=1, keepdims=True)
        return distance, far, acc

    _, _, acc = lax.fori_loop(
        0, NPOINT, body,
        (jnp.full((b, n), 1e10, _F32), jnp.zeros((b, 1), _I32),
         jnp.zeros((kb, NPOINT), _F32)))
    out_ref[...] = acc


def _bqpre_body(xyzt_ref, new_ref, feat_ref, w0f_ref, w0x_ref, b0_ref,
                idx_ref, tab_ref):
    n = xyzt_ref.shape[2]
    bidx = pl.program_id(0)
    x3n = xyzt_ref[0]
    nb = new_ref[0]

    a = lax.dot_general(feat_ref[0], w0f_ref[...], (((0,), (1,)), ((), ())),
                        preferred_element_type=_F32)
    a = a + lax.dot_general(x3n * _F32(1.0 / RADIUS), w0x_ref[...],
                            (((0,), (1,)), ((), ())),
                            preferred_element_type=_F32)
    tab_ref[0] = a + b0_ref[...]

    d2 = jnp.zeros((NPOINT, n), _F32)
    for k in range(3):
        diff = nb[:, k:k + 1] - x3n[k:k + 1, :]
        d2 = d2 + diff * diff
    lane = lax.broadcasted_iota(_I32, (NPOINT, n), 1)
    lane_s = lax.broadcasted_iota(_I32, (NPOINT, NSAMPLE), 1)
    work = jnp.where(d2 < _F32(RADIUS * RADIUS), lane, n)
    first0 = jnp.min(work, axis=1, keepdims=True)
    pad0 = jnp.where(first0 == n, 0, first0)
    acc = jnp.broadcast_to(pad0, (NPOINT, NSAMPLE))
    work = jnp.where(work == first0, n, work)

    def cond(st):
        return st[3]

    def body(st):
        s, work, acc, _ = st
        first = jnp.min(work, axis=1, keepdims=True)
        acc = jnp.where(jnp.logical_and(lane_s == s, first < n), first, acc)
        work = jnp.where(work == first, n, work)
        alive = jnp.logical_and(s + 1 < NSAMPLE, jnp.min(first) < n)
        return (s + 1, work, acc, alive)

    alive0 = jnp.logical_and(1 < NSAMPLE, jnp.min(work) < n)
    _, _, acc, _ = lax.while_loop(
        cond, body, (jnp.int32(1), work, acc, alive0))
    idx_ref[0] = acc + bidx * n


def _head_body(grp_ref, new_ref, w0x_ref,
               sw1_ref, sb1_ref, sw2_ref, sb2_ref,
               hw1_ref, hb1_ref, hw2_ref, hb2_ref,
               hw3_ref, hb3_ref, scale_ref, esel_ref, out_ref):
    def dg(x, w):
        return lax.dot_general(x, w, (((1,), (1,)), ((), ())),
                               preferred_element_type=_F32)

    g = grp_ref[0]
    nx = new_ref[0]
    v = dg(nx * _F32(1.0 / RADIUS), w0x_ref[...])
    vrep = jnp.broadcast_to(
        v[:, None, :], (NPOINT, NSAMPLE, 128)).reshape(NPOINT * NSAMPLE, 128)
    h = jnp.maximum(g - vrep, 0.0)
    h = jnp.maximum(dg(h, sw1_ref[...]) + sb1_ref[...], 0.0)
    h = jnp.maximum(dg(h, sw2_ref[...]) + sb2_ref[...], 0.0)
    pooled = jnp.max(h.reshape(NPOINT, NSAMPLE, 128), axis=1)
    n1 = jnp.maximum(dg(pooled, hw1_ref[...]) + hb1_ref[...], 0.0)
    n2 = jnp.maximum(dg(n1, hw2_ref[...]) + hb2_ref[...], 0.0)
    nt = dg(n2, hw3_ref[...]) + hb3_ref[...]
    out_ref[0] = nt * scale_ref[...] + jnp.dot(
        nx, esel_ref[...], preferred_element_type=_F32)


def _sc_gather(table, idx2d):
    rows_tot = idx2d.shape[0]
    n_workers = 32
    per_w = rows_tot // n_workers
    nb = 3
    dt = table.dtype
    mesh = plsc.VectorSubcoreMesh(core_axis_name="c", subcore_axis_name="s")

    @functools.partial(
        pl.kernel,
        out_type=jax.ShapeDtypeStruct((rows_tot * 128, 128), dt),
        mesh=mesh,
        scratch_types=(
            [pltpu.VMEM((per_w, 128), _I32)]
            + [pltpu.VMEM((128, 128), dt) for _ in range(nb)]
            + [pltpu.SemaphoreType.DMA for _ in range(2 * nb)]
        ),
    )
    def gk(table_hbm, idx_hbm, out_hbm, idx_v, *bs):
        bufs = bs[:nb]
        gsem = bs[nb:2 * nb]
        ssem = bs[2 * nb:]
        cid = lax.axis_index("c")
        sid = lax.axis_index("s")
        wid = sid * 2 + cid
        base = wid * per_w
        pltpu.sync_copy(idx_hbm.at[pl.ds(base, per_w)], idx_v)
        gcp = [None] * per_w
        scp = [None] * per_w

        def fire_store(j):
            scp[j] = pltpu.async_copy(
                bufs[j % nb],
                out_hbm.at[pl.ds((base + j) * 128, 128)], ssem[j % nb])

        for j in range(per_w):
            if j >= nb:
                scp[j - nb].wait()
            gcp[j] = pltpu.async_copy(
                table_hbm.at[idx_v.at[j]], bufs[j % nb], gsem[j % nb])
            if j > 0:
                gcp[j - 1].wait()
                fire_store(j - 1)
        gcp[per_w - 1].wait()
        fire_store(per_w - 1)
        for j in range(max(per_w - nb, 0), per_w):
            scp[j].wait()

    return gk(table, idx2d)


_ESEL = np.zeros((3, OUT_CH), np.float32)
_ESEL[0, 2] = 1.0
_ESEL[1, 3] = 1.0
_ESEL[2, 4] = 1.0


def kernel(xyz, features, sa_w0, sa_b0, sa_g0, sa_be0, sa_w1, sa_b1, sa_g1,
           sa_be1, sa_w2, sa_b2, sa_g2, sa_be2, w1, b1, g1, be1, w2, b2, g2,
           be2, w3, b3, mean_size_arr):
    b, n, _ = xyz.shape
    c = features.shape[1]

    w0e = sa_w0 * sa_g0[:, None]
    b0e = sa_b0 * sa_g0 + sa_be0
    w0x = w0e[:, :3]
    w0f = w0e[:, 3:]
    sw1 = sa_w1 * sa_g1[:, None]
    sb1 = sa_b1 * sa_g1 + sa_be1
    sw2 = sa_w2 * sa_g2[:, None]
    sb2 = sa_b2 * sa_g2 + sa_be2
    hw1 = w1 * g1[:, None]
    hb1 = b1 * g1 + be1
    hw2 = w2 * g2[:, None]
    hb2 = b2 * g2 + be2

    xyzt = jnp.transpose(xyz, (2, 0, 1)).reshape(3 * b, n)
    new_seq = pl.pallas_call(
        _fps_body,
        out_shape=jax.ShapeDtypeStruct((3 * b, NPOINT), _F32),
    )(xyzt)
    new_xyz = jnp.transpose(new_seq.reshape(3, b, NPOINT), (1, 2, 0))

    xyzt_b = jnp.transpose(xyz, (0, 2, 1))
    hb = b // 2

    def run_bqpre(h):
        sl = slice(h * hb, (h + 1) * hb)
        return pl.pallas_call(
            _bqpre_body,
            grid=(hb,),
            in_specs=[
                pl.BlockSpec((1, 3, n), lambda i: (i, 0, 0)),
                pl.BlockSpec((1, NPOINT, 3), lambda i: (i, 0, 0)),
                pl.BlockSpec((1, c, n), lambda i: (i, 0, 0)),
                pl.BlockSpec((128, c), lambda i: (0, 0)),
                pl.BlockSpec((128, 3), lambda i: (0, 0)),
                pl.BlockSpec((1, 128), lambda i: (0, 0)),
            ],
            out_specs=[
                pl.BlockSpec((1, NPOINT, NSAMPLE), lambda i: (i, 0, 0)),
                pl.BlockSpec((1, n, 128), lambda i: (i, 0, 0)),
            ],
            out_shape=[
                jax.ShapeDtypeStruct((hb, NPOINT, NSAMPLE), _I32),
                jax.ShapeDtypeStruct((hb, n, 128), _F32),
            ],
        )(xyzt_b[sl], new_xyz[sl], features[sl], w0f, w0x,
          b0e.reshape(1, 128))

    grouped_halves = []
    for h in range(2):
        idx_h, table_h = run_bqpre(h)
        grouped_halves.append(
            _sc_gather(table_h.reshape(hb * n, 128), idx_h.reshape(-1, 128))
            .reshape(hb, NPOINT * NSAMPLE, 128))

    msa_flat = mean_size_arr.reshape(-1)
    scale = jnp.concatenate([
        jnp.ones((5,), _F32),
        jnp.ones((NUM_HEADING_BIN,), _F32),
        jnp.full((NUM_HEADING_BIN,), np.pi / NUM_HEADING_BIN, _F32),
        jnp.ones((NUM_SIZE_CLUSTER,), _F32),
        msa_flat.astype(_F32),
        jnp.ones((NUM_CLASS,), _F32),
    ]).reshape(1, OUT_CH)
    esel = jnp.asarray(_ESEL)

    full = lambda s: pl.BlockSpec(s, lambda i: tuple(0 for _ in s))
    row = lambda: pl.BlockSpec((1, 128), lambda i: (0, 0))

    def run_head(grouped_h, new_h):
        bh = grouped_h.shape[0]
        return pl.pallas_call(
            _head_body,
            grid=(bh,),
            in_specs=[
                pl.BlockSpec((1, NPOINT * NSAMPLE, 128), lambda i: (i, 0, 0)),
                pl.BlockSpec((1, NPOINT, 3), lambda i: (i, 0, 0)),
                full((128, 3)),
                full((128, 128)), row(),
                full((128, 128)), row(),
                full((128, 128)), row(),
                full((128, 128)), row(),
                full((OUT_CH, 128)),
                pl.BlockSpec((1, OUT_CH), lambda i: (0, 0)),
                pl.BlockSpec((1, OUT_CH), lambda i: (0, 0)),
                pl.BlockSpec((3, OUT_CH), lambda i: (0, 0)),
            ],
            out_specs=pl.BlockSpec((1, NPOINT, OUT_CH), lambda i: (i, 0, 0)),
            out_shape=jax.ShapeDtypeStruct((bh, NPOINT, OUT_CH), _F32),
        )(grouped_h, new_h, w0x,
          sw1, sb1.reshape(1, 128),
          sw2, sb2.reshape(1, 128),
          hw1, hb1.reshape(1, 128),
          hw2, hb2.reshape(1, 128),
          w3, b3.reshape(1, OUT_CH), scale, esel)

    out_halves = [
        run_head(grouped_halves[h], new_xyz[h * hb:(h + 1) * hb])
        for h in range(2)
    ]
    return jnp.concatenate(out_halves, axis=0)
